# baseline (speedup 1.0000x reference)
import functools

import jax
import jax.numpy as jnp
from jax import lax
from jax.experimental import pallas as pl
from jax.experimental.pallas import tpu as pltpu
from jax.experimental.pallas import tpu_sc as plsc

_NUM_Q = 100000
_D = 64
_B, _T = 1024, 200
_N = _B * _T
_NC, _NS = 2, 16
_NW = _NC * _NS
_NT = _N // _NW
_C = 400
_G = _NT // _C
_MAX_IID = 2 * _NUM_Q
_EPS = 1e-5


def _tec_body(aux_hbm, qtab_hbm, itab_hbm,
              x_hbm, qemb_hbm,
              qspan_v, ispan_v, qr0, ir0, qr1, ir1, gb_v,
              g0, g1, wq0, wq1):
    cid = lax.axis_index("c")
    sid = lax.axis_index("s")
    wid = sid * _NC + cid
    span0 = pl.multiple_of(wid * _NT, _NT)
    pltpu.sync_copy(aux_hbm.at[pl.ds(2 * _N, 2 * _D)], gb_v)
    pltpu.sync_copy(aux_hbm.at[pl.ds(span0, _NT)], qspan_v)
    pltpu.sync_copy(aux_hbm.at[pl.ds(_N + span0, _NT)], ispan_v)

    def idx_row(r, c0):
        s = pl.ds(r * 16, 16)
        q = qspan_v[s]
        rr = ispan_v[s]
        rr = jnp.minimum(jnp.maximum(rr, 0), 1)
        iid = q + q + rr
        iid = jnp.where(q == 0, 0, iid)
        ispan_v[s] = jnp.minimum(iid, _MAX_IID)
        return c0

    lax.fori_loop(0, _NT // 16, idx_row, 0)

    bufs = ((qr0, ir0, g0, wq0), (qr1, ir1, g1, wq1))

    def issue_gathers(c, qr, ir, sem):
        off = c * _C
        pltpu.async_copy(qtab_hbm.at[qspan_v.at[pl.ds(off, _C)]], qr, sem)
        pltpu.async_copy(itab_hbm.at[ispan_v.at[pl.ds(off, _C)]], ir, sem)

    def drain_gathers(qr, ir, sem):
        pltpu.make_async_copy(qtab_hbm.at[qspan_v.at[pl.ds(0, _C)]], qr,
                              sem).wait()
        pltpu.make_async_copy(itab_hbm.at[ispan_v.at[pl.ds(0, _C)]], ir,
                              sem).wait()

    inv_d = jnp.float32(1.0 / _D)
    lanes = lax.iota(jnp.int32, 16)
    gams = [plsc.bitcast(gb_v[pl.ds(j * 16, 16)], jnp.float32)
            for j in range(_D // 16)]
    bets = [plsc.bitcast(gb_v[pl.ds(_D + j * 16, 16)], jnp.float32)
            for j in range(_D // 16)]

    def hsum(v):
        for k in (8, 4, 2, 1):
            v = v + v.at[lanes ^ k].get(mode="promise_in_bounds")
        return v

    _U = 4

    def make_tok(qr, ir):
        def tok(i, c2):
            for u in range(_U):
                t = i * _U + u
                v = []
                for j in range(_D // 16):
                    s = pl.ds(j * 16, 16)
                    v.append(qr[t, s] + ir[t, s])
                stot = (v[0] + v[1]) + (v[2] + v[3])
                sstot = (v[0] * v[0] + v[1] * v[1]) + (
                    v[2] * v[2] + v[3] * v[3])
                mv = hsum(stot) * inv_d
                vv = hsum(sstot) * inv_d - mv * mv + jnp.float32(_EPS)
                iy = jnp.int32(0x5F3759DF) - lax.shift_right_logical(
                    plsc.bitcast(vv, jnp.int32), 1)
                y = plsc.bitcast(iy, jnp.float32)
                for _ in range(2):
                    y = y * (jnp.float32(1.5) - jnp.float32(0.5) * vv * y * y)
                nb = mv * y
                for j in range(_D // 16):
                    s = pl.ds(j * 16, 16)
                    ir[t, s] = (v[j] * y - nb) * gams[j] + bets[j]
            return c2
        return tok

    def half(b, c):
        qr, ir, gsem, wqsem = bufs[b]
        qro, iro, gsemo, wqsemo = bufs[1 - b]

        @pl.when(c + 1 < _G)
        def _():
            @pl.when(c > 0)
            def _():
                pltpu.make_async_copy(
                    qro.at[pl.ds(0, _T)], qemb_hbm.at[0], wqsemo).wait()
                pltpu.make_async_copy(
                    qro.at[pl.ds(_T, _T)], qemb_hbm.at[0], wqsemo).wait()
            issue_gathers(c + 1, qro, iro, gsemo)

        drain_gathers(qr, ir, gsem)
        b0 = wid * (_NT // _T) + c * (_C // _T)
        pltpu.async_copy(qr.at[pl.ds(0, _T)], qemb_hbm.at[b0], wqsem)
        pltpu.async_copy(qr.at[pl.ds(_T, _T)], qemb_hbm.at[b0 + 1], wqsem)
        lax.fori_loop(0, _C // _U, make_tok(qr, ir), 0)
        pltpu.sync_copy(ir.at[pl.ds(0, _T)], x_hbm.at[b0])
        pltpu.sync_copy(ir.at[pl.ds(_T, _T)], x_hbm.at[b0 + 1])

    issue_gathers(0, qr0, ir0, g0)

    def pair(i, carry):
        half(0, 2 * i)
        half(1, 2 * i + 1)
        return carry

    lax.fori_loop(0, _G // 2, pair, 0)
    pltpu.make_async_copy(qr0.at[pl.ds(0, _T)], qemb_hbm.at[0], wq0).wait()
    pltpu.make_async_copy(qr0.at[pl.ds(_T, _T)], qemb_hbm.at[0], wq0).wait()
    pltpu.make_async_copy(qr1.at[pl.ds(0, _T)], qemb_hbm.at[0], wq1).wait()
    pltpu.make_async_copy(qr1.at[pl.ds(_T, _T)], qemb_hbm.at[0], wq1).wait()


def kernel(question_ids, responses, question_table, interaction_table,
           ln_gamma, ln_beta):
    qid = question_ids.reshape(_N).astype(jnp.int32)
    resp = responses.reshape(_N).astype(jnp.int32)
    aux = jnp.concatenate([
        qid, resp,
        lax.bitcast_convert_type(ln_gamma, jnp.int32),
        lax.bitcast_convert_type(ln_beta, jnp.int32),
    ])
    mesh = plsc.VectorSubcoreMesh(core_axis_name="c", subcore_axis_name="s")
    run = pl.kernel(
        _tec_body,
        out_type=(
            jax.ShapeDtypeStruct((_B, _T, _D), jnp.float32),
            jax.ShapeDtypeStruct((_B, _T, _D), jnp.float32),
        ),
        mesh=mesh,
        compiler_params=pltpu.CompilerParams(
            needs_layout_passes=False, use_tc_tiling_on_sc=False),
        scratch_types=[
            pltpu.VMEM((_NT,), jnp.int32),
            pltpu.VMEM((_NT,), jnp.int32),
            pltpu.VMEM((_C, _D), jnp.float32),
            pltpu.VMEM((_C, _D), jnp.float32),
            pltpu.VMEM((_C, _D), jnp.float32),
            pltpu.VMEM((_C, _D), jnp.float32),
            pltpu.VMEM((2 * _D,), jnp.int32),
            pltpu.SemaphoreType.DMA,
            pltpu.SemaphoreType.DMA,
            pltpu.SemaphoreType.DMA,
            pltpu.SemaphoreType.DMA,
        ],
    )
    x, qemb = run(aux, question_table, interaction_table)
    return (x, qemb)


# speedup vs baseline: 1.0964x; 1.0964x over previous
import functools

import jax
import jax.numpy as jnp
from jax import lax
from jax.experimental import pallas as pl
from jax.experimental.pallas import tpu as pltpu
from jax.experimental.pallas import tpu_sc as plsc

_NUM_Q = 100000
_D = 64
_B, _T = 1024, 200
_N = _B * _T
_NC, _NS = 2, 16
_NW = _NC * _NS
_NT = _N // _NW
_C = _T
_G = _NT // _C
_MAX_IID = 2 * _NUM_Q
_EPS = 1e-5


def _tec_body(aux_hbm, qtab_hbm, itab_hbm,
              x_hbm, qemb_hbm,
              qspan_v, ispan_v, qr0, ir0, xb0, qr1, ir1, xb1, gb_v,
              g0, g1, wq0, wq1, wx0, wx1):
    cid = lax.axis_index("c")
    sid = lax.axis_index("s")
    wid = sid * _NC + cid
    span0 = pl.multiple_of(wid * _NT, _NT)
    pltpu.sync_copy(aux_hbm.at[pl.ds(2 * _N, 2 * _D)], gb_v)
    pltpu.sync_copy(aux_hbm.at[pl.ds(span0, _NT)], qspan_v)
    pltpu.sync_copy(aux_hbm.at[pl.ds(_N + span0, _NT)], ispan_v)

    def idx_row(r, c0):
        s = pl.ds(r * 16, 16)
        q = qspan_v[s]
        rr = ispan_v[s]
        rr = jnp.minimum(jnp.maximum(rr, 0), 1)
        iid = q + q + rr
        iid = jnp.where(q == 0, 0, iid)
        ispan_v[s] = jnp.minimum(iid, _MAX_IID)
        return c0

    lax.fori_loop(0, _NT // 16, idx_row, 0)

    bufs = ((qr0, ir0, xb0, g0, wq0, wx0), (qr1, ir1, xb1, g1, wq1, wx1))

    def issue_gathers(c, qr, ir, sem):
        off = c * _C
        pltpu.async_copy(qtab_hbm.at[qspan_v.at[pl.ds(off, _C)]], qr, sem)
        pltpu.async_copy(itab_hbm.at[ispan_v.at[pl.ds(off, _C)]], ir, sem)

    def drain_gathers(qr, ir, sem):
        pltpu.make_async_copy(qtab_hbm.at[qspan_v.at[pl.ds(0, _C)]], qr,
                              sem).wait()
        pltpu.make_async_copy(itab_hbm.at[ispan_v.at[pl.ds(0, _C)]], ir,
                              sem).wait()

    inv_d = jnp.float32(1.0 / _D)
    lanes = lax.iota(jnp.int32, 16)
    gams = [plsc.bitcast(gb_v[pl.ds(j * 16, 16)], jnp.float32)
            for j in range(_D // 16)]
    bets = [plsc.bitcast(gb_v[pl.ds(_D + j * 16, 16)], jnp.float32)
            for j in range(_D // 16)]

    def hsum(v):
        for k in (8, 4, 2, 1):
            v = v + v.at[lanes ^ k].get(mode="promise_in_bounds")
        return v

    _U = 4

    def make_tok(qr, ir, xb):
        def tok(i, c2):
            for u in range(_U):
                t = i * _U + u
                v = []
                for j in range(_D // 16):
                    s = pl.ds(j * 16, 16)
                    v.append(qr[t, s] + ir[t, s])
                stot = (v[0] + v[1]) + (v[2] + v[3])
                sstot = (v[0] * v[0] + v[1] * v[1]) + (
                    v[2] * v[2] + v[3] * v[3])
                mv = hsum(stot) * inv_d
                vv = hsum(sstot) * inv_d - mv * mv + jnp.float32(_EPS)
                iy = jnp.int32(0x5F3759DF) - lax.shift_right_logical(
                    plsc.bitcast(vv, jnp.int32), 1)
                y = plsc.bitcast(iy, jnp.float32)
                for _ in range(2):
                    y = y * (jnp.float32(1.5) - jnp.float32(0.5) * vv * y * y)
                nb = mv * y
                for j in range(_D // 16):
                    s = pl.ds(j * 16, 16)
                    xb[t, s] = (v[j] * y - nb) * gams[j] + bets[j]
            return c2
        return tok

    def half(b, c):
        qr, ir, xb, gsem, wqsem, wxsem = bufs[b]
        qro, iro, xbo, gsemo, wqsemo, wxsemo = bufs[1 - b]

        @pl.when(c + 1 < _G)
        def _():
            @pl.when(c > 0)
            def _():
                pltpu.make_async_copy(qro, qemb_hbm.at[0], wqsemo).wait()
            issue_gathers(c + 1, qro, iro, gsemo)

        drain_gathers(qr, ir, gsem)
        b0 = wid * _G + c
        pltpu.async_copy(qr, qemb_hbm.at[b0], wqsem)

        @pl.when(c > 1)
        def _():
            pltpu.make_async_copy(xb, x_hbm.at[0], wxsem).wait()

        lax.fori_loop(0, _C // _U, make_tok(qr, ir, xb), 0)
        pltpu.async_copy(xb, x_hbm.at[b0], wxsem)

    issue_gathers(0, qr0, ir0, g0)

    def pair(i, carry):
        half(0, 2 * i)
        half(1, 2 * i + 1)
        return carry

    lax.fori_loop(0, _G // 2, pair, 0)
    pltpu.make_async_copy(qr0, qemb_hbm.at[0], wq0).wait()
    pltpu.make_async_copy(qr1, qemb_hbm.at[0], wq1).wait()
    pltpu.make_async_copy(xb0, x_hbm.at[0], wx0).wait()
    pltpu.make_async_copy(xb1, x_hbm.at[0], wx1).wait()


def kernel(question_ids, responses, question_table, interaction_table,
           ln_gamma, ln_beta):
    qid = question_ids.reshape(_N).astype(jnp.int32)
    resp = responses.reshape(_N).astype(jnp.int32)
    aux = jnp.concatenate([
        qid, resp,
        lax.bitcast_convert_type(ln_gamma, jnp.int32),
        lax.bitcast_convert_type(ln_beta, jnp.int32),
    ])
    mesh = plsc.VectorSubcoreMesh(core_axis_name="c", subcore_axis_name="s")
    run = pl.kernel(
        _tec_body,
        out_type=(
            jax.ShapeDtypeStruct((_B, _T, _D), jnp.float32),
            jax.ShapeDtypeStruct((_B, _T, _D), jnp.float32),
        ),
        mesh=mesh,
        compiler_params=pltpu.CompilerParams(
            needs_layout_passes=False, use_tc_tiling_on_sc=False),
        scratch_types=[
            pltpu.VMEM((_NT,), jnp.int32),
            pltpu.VMEM((_NT,), jnp.int32),
            pltpu.VMEM((_C, _D), jnp.float32),
            pltpu.VMEM((_C, _D), jnp.float32),
            pltpu.VMEM((_C, _D), jnp.float32),
            pltpu.VMEM((_C, _D), jnp.float32),
            pltpu.VMEM((_C, _D), jnp.float32),
            pltpu.VMEM((_C, _D), jnp.float32),
            pltpu.VMEM((2 * _D,), jnp.int32),
            pltpu.SemaphoreType.DMA,
            pltpu.SemaphoreType.DMA,
            pltpu.SemaphoreType.DMA,
            pltpu.SemaphoreType.DMA,
            pltpu.SemaphoreType.DMA,
            pltpu.SemaphoreType.DMA,
        ],
    )
    x, qemb = run(aux, question_table, interaction_table)
    return (x, qemb)
